# Initial kernel scaffold; baseline (speedup 1.0000x reference)
import jax
import jax.numpy as jnp
from jax.experimental import pallas as pl


def kernel(user_indices, item_indices, edge_index_t0, weights_t0, edge_index_t1, weights_t1, user_emb, item_emb, type_weights):
    raise NotImplementedError("write your pallas kernel here")


# speedup vs baseline: 2.1068x; 2.1068x over previous
import functools

import jax
import jax.numpy as jnp
from jax import lax
from jax.experimental import pallas as pl
from jax.experimental.pallas import tpu as pltpu
from jax.experimental.pallas import tpu_sc as plsc

NU = 50000
NI = 100000
D = 64
DH = 32
E2 = 1048576
NC, NS, L = 2, 16, 16
CH = 128
NUP = 50048
ROWS_PER_SUB = NUP // NS
ZCH = 136


def _mesh():
    return plsc.VectorSubcoreMesh(core_axis_name="c", subcore_axis_name="s")


def _make_spmm(n_in_rows, a_mult, c_mult):
    nchunks = E2 // NS // CH

    @functools.partial(
        pl.kernel,
        mesh=_mesh(),
        compiler_params=pltpu.CompilerParams(use_tc_tiling_on_sc=False),
        out_type=jax.ShapeDtypeStruct((2 * NUP, DH), jnp.float32),
        scratch_types=[
            pltpu.VMEM_SHARED((NUP, DH), jnp.float32),
            pltpu.VMEM((CH,), jnp.int32),
            pltpu.VMEM((CH,), jnp.int32),
            pltpu.VMEM((CH,), jnp.float32),
            pltpu.VMEM((CH, DH), jnp.float32),
            pltpu.VMEM((ZCH, DH), jnp.float32),
            pltpu.SemaphoreType.DMA,
        ],
    )
    def spmm(table, graw, sraw, w, out, acc, gidx_v, sidx_v, w_v, rows_v,
             zb_v, sem):
        c = lax.axis_index("c")
        s = lax.axis_index("s")

        z16 = jnp.zeros((L,), jnp.float32)

        def zrow(j, _):
            zb_v[j, pl.ds(0, L)] = z16
            zb_v[j, pl.ds(L, L)] = z16
            return 0

        lax.fori_loop(0, ZCH, zrow, 0)

        def zcopy(t, _):
            pltpu.sync_copy(zb_v, acc.at[pl.ds(s * ROWS_PER_SUB + t * ZCH, ZCH)])
            return 0

        lax.fori_loop(0, ROWS_PER_SUB // ZCH, zcopy, 0)
        plsc.subcore_barrier()

        base = s * (E2 // NS)
        cmul = (c * c_mult).astype(jnp.int32)
        cvec = jnp.full((L,), cmul, jnp.int32)

        def chunk(k, _):
            off = base + k * CH
            pltpu.sync_copy(graw.at[pl.ds(off, CH)], gidx_v)
            pltpu.sync_copy(sraw.at[pl.ds(off, CH)], sidx_v)
            pltpu.sync_copy(w.at[pl.ds(off, CH)], w_v)

            def fix(j, _):
                g = gidx_v[pl.ds(j * L, L)]
                gidx_v[pl.ds(j * L, L)] = g * a_mult + cvec
                return 0

            lax.fori_loop(0, CH // L, fix, 0)

            pltpu.async_copy(table.at[gidx_v], rows_v, sem).wait()

            def scale(j, _):
                w16 = w_v[pl.ds(j * L, L)]
                for l in range(L):
                    e = j * L + l
                    ws = jnp.full((L,), w16[l], jnp.float32)
                    rows_v[e, pl.ds(0, L)] = rows_v[e, pl.ds(0, L)] * ws
                    rows_v[e, pl.ds(L, L)] = rows_v[e, pl.ds(L, L)] * ws
                return 0

            lax.fori_loop(0, CH // L, scale, 0)

            pltpu.sync_copy(rows_v, acc.at[sidx_v], add=True)
            return 0

        lax.fori_loop(0, nchunks, chunk, 0)
        plsc.subcore_barrier()

        r0 = s * ROWS_PER_SUB
        pltpu.sync_copy(acc.at[pl.ds(r0, ROWS_PER_SUB)],
                        out.at[pl.ds(c * NUP + r0, ROWS_PER_SUB)])

    return spmm


def _make_tail(B):
    npb = B // (NC * NS)
    nch = npb // CH

    rowbuf = lambda: pltpu.VMEM((CH, DH), jnp.float32)

    @functools.partial(
        pl.kernel,
        mesh=_mesh(),
        compiler_params=pltpu.CompilerParams(use_tc_tiling_on_sc=False),
        out_type=jax.ShapeDtypeStruct((B,), jnp.float32),
        scratch_types=[
            pltpu.VMEM((CH,), jnp.int32),
            pltpu.VMEM((CH,), jnp.int32),
            pltpu.VMEM((CH,), jnp.int32),
            pltpu.VMEM((CH,), jnp.int32),
            pltpu.VMEM((CH,), jnp.int32),
            pltpu.VMEM((CH,), jnp.int32),
            rowbuf(), rowbuf(), rowbuf(), rowbuf(), rowbuf(),
            rowbuf(), rowbuf(), rowbuf(), rowbuf(), rowbuf(),
            pltpu.VMEM((CH,), jnp.float32),
            pltpu.VMEM((L,), jnp.float32),
            pltpu.SemaphoreType.DMA,
        ],
    )
    def tail(u0, i0, u1a, u2a, u1b, u2b, i1a, i2a, i1b, i2b, uidx, iidx,
             tw16, out, ui_v, ii_v, ga_v, gb_v, gc_v, gd_v,
             bu0, bu1a, bu2a, bu1b, bu2b, bi0, bi1a, bi2a, bi1b, bi2b,
             dot_v, sm_v, sem):
        c = lax.axis_index("c")
        s = lax.axis_index("s")
        wid = c * NS + s

        pltpu.sync_copy(tw16, sm_v)
        sv = sm_v[pl.ds(0, L)]
        t0 = sv[0]
        t1 = sv[1]
        m = jnp.maximum(t0, t1)
        ev = jnp.exp(sv - jnp.full((L,), m, jnp.float32))
        e0 = ev[0]
        e1 = ev[1]
        wv = ev / jnp.full((L,), e0 + e1, jnp.float32)
        w0 = wv[0]
        w1 = wv[1]

        limit = jnp.full((L,), NU - 1, jnp.int32)
        lane = lax.iota(jnp.int32, L)
        perms = [jnp.bitwise_xor(lane, k) for k in (8, 4, 2, 1)]

        gdn = lax.GatherDimensionNumbers(offset_dims=(), collapsed_slice_dims=(0,),
                                         start_index_map=(0,))

        def _hsum(v):
            for p in perms:
                v = v + lax.gather(v, p[:, None], gdn, (1,),
                                   mode=lax.GatherScatterMode.PROMISE_IN_BOUNDS)
            return v

        def chunk(ch, _):
            ob = wid * npb + ch * CH
            pltpu.sync_copy(uidx.at[pl.ds(ob, CH)], ui_v)
            pltpu.sync_copy(iidx.at[pl.ds(ob, CH)], ii_v)

            def half(c01):
                coff = jnp.full((L,), c01, jnp.int32)
                cblk = jnp.full((L,), c01 * NUP, jnp.int32)

                def fix(j, _):
                    u = ui_v[pl.ds(j * L, L)]
                    i = ii_v[pl.ds(j * L, L)]
                    ga_v[pl.ds(j * L, L)] = u * 2 + coff
                    gb_v[pl.ds(j * L, L)] = u + cblk
                    gc_v[pl.ds(j * L, L)] = i * 2 + coff
                    gd_v[pl.ds(j * L, L)] = jnp.minimum(i, limit) + cblk
                    return 0

                lax.fori_loop(0, CH // L, fix, 0)

                cps = [
                    pltpu.async_copy(u0.at[ga_v], bu0, sem),
                    pltpu.async_copy(u1a.at[gb_v], bu1a, sem),
                    pltpu.async_copy(u2a.at[gb_v], bu2a, sem),
                    pltpu.async_copy(u1b.at[gb_v], bu1b, sem),
                    pltpu.async_copy(u2b.at[gb_v], bu2b, sem),
                    pltpu.async_copy(i0.at[gc_v], bi0, sem),
                    pltpu.async_copy(i1a.at[gd_v], bi1a, sem),
                    pltpu.async_copy(i2a.at[gd_v], bi2a, sem),
                    pltpu.async_copy(i1b.at[gd_v], bi1b, sem),
                    pltpu.async_copy(i2b.at[gd_v], bi2b, sem),
                ]
                for cp in cps:
                    cp.wait()

                b0 = jnp.full((L,), w0, jnp.float32)
                b1 = jnp.full((L,), w1, jnp.float32)

                def body(j, _):
                    ii16 = ii_v[pl.ds(j * L, L)]
                    dots = jnp.zeros((L,), jnp.float32)
                    for l in range(L):
                        e = j * L + l
                        vf = jnp.where(ii16[l] < NU, 1.0, 0.0)
                        a0 = jnp.full((L,), w0 * vf, jnp.float32)
                        a1 = jnp.full((L,), w1 * vf, jnp.float32)
                        acc = jnp.zeros((L,), jnp.float32)
                        for lo in (0, L):
                            sl = pl.ds(lo, L)
                            uu = (bu0[e, sl] + b0 * (bu1a[e, sl] + bu2a[e, sl])
                                  + b1 * (bu1b[e, sl] + bu2b[e, sl]))
                            iv = (bi0[e, sl] + a0 * (bi1a[e, sl] + bi2a[e, sl])
                                  + a1 * (bi1b[e, sl] + bi2b[e, sl]))
                            acc = acc + uu * iv
                        vs = _hsum(acc) * (1.0 / 9.0)
                        dots = jnp.where(lane == l, vs, dots)
                    sl16 = pl.ds(j * L, L)
                    if c01 == 0:
                        dot_v[sl16] = dots
                    else:
                        dot_v[sl16] = dot_v[sl16] + dots
                    return 0

                lax.fori_loop(0, CH // L, body, 0)

            half(0)
            half(1)
            pltpu.sync_copy(dot_v, out.at[pl.ds(ob, CH)])
            return 0

        lax.fori_loop(0, nch, chunk, 0)

    return tail


def _pad_edges(idx_row, n):
    return jnp.concatenate([idx_row, jnp.zeros((n,), idx_row.dtype)])


def kernel(user_indices, item_indices, edge_index_t0, weights_t0,
           edge_index_t1, weights_t1, user_emb, item_emb, type_weights):
    E = edge_index_t0.shape[1]
    npad = E2 - E
    B = user_indices.shape[0]

    u0v = user_emb.reshape(2 * NU, DH)
    i0v = item_emb.reshape(2 * NI, DH)

    spmm_u0 = _make_spmm(2 * NU, 2, 1)
    spmm_i0 = _make_spmm(2 * NI, 2, 1)
    spmm_mid = _make_spmm(2 * NUP, 1, NUP)

    outs = []
    for ei, w in ((edge_index_t0, weights_t0), (edge_index_t1, weights_t1)):
        src = _pad_edges(ei[0], npad)
        dst = _pad_edges(ei[1], npad)
        wp = jnp.concatenate([w, jnp.zeros((npad,), w.dtype)])
        u1 = spmm_i0(i0v, dst, src, wp)
        i1 = spmm_u0(u0v, src, dst, wp)
        u2 = spmm_mid(i1, dst, src, wp)
        i2 = spmm_mid(u1, src, dst, wp)
        outs.append((u1, u2, i1, i2))

    (u1a, u2a, i1a, i2a), (u1b, u2b, i1b, i2b) = outs
    tw16 = jnp.pad(type_weights.astype(jnp.float32), (0, L - 2))

    tail = _make_tail(B)
    return tail(u0v, i0v, u1a, u2a, u1b, u2b, i1a, i2a, i1b, i2b,
                user_indices.astype(jnp.int32), item_indices.astype(jnp.int32),
                tw16)
